# baseline (speedup 1.0000x reference)
import functools

import jax
import jax.numpy as jnp
from jax import lax
from jax.experimental import pallas as pl

B, C, N = 8, 128, 1024
K = 20
K2 = 14
TN = 128
HIGHEST = lax.Precision.HIGHEST


def _knn_body(xt_tile_ref, xt_full_ref, x_full_ref, idx_ref, x1_ref):
    xt_tile = xt_tile_ref[0]
    xt_full = xt_full_ref[0]
    x_full = x_full_ref[0]

    xx = jnp.sum(x_full * x_full, axis=0, keepdims=True)
    dist = 2.0 * jnp.dot(xt_tile, x_full) - xx

    lane_iota = lax.broadcasted_iota(jnp.int32, (TN, N), 1)
    neg_inf = jnp.float32(-jnp.inf)

    idx_cols = []
    for _ in range(K):
        m = jnp.max(dist, axis=1, keepdims=True)
        amax = jnp.min(jnp.where(dist == m, lane_iota, N),
                       axis=1, keepdims=True)
        idx_cols.append(amax)
        dist = jnp.where(lane_iota == amax, neg_inf, dist)
    idx_tile = jnp.concatenate(idx_cols, axis=1)
    idx_ref[0] = idx_tile

    oh_all = (lax.broadcasted_iota(jnp.int32, (TN, K, N), 2)
              == idx_tile[:, :, None]).astype(jnp.float32).reshape(TN * K, N)
    gall = jnp.dot(oh_all, xt_full, precision=HIGHEST)
    knn = gall.reshape(TN, K, C)
    s20 = jnp.sum(knn, axis=1)

    kk_iota = lax.broadcasted_iota(jnp.int32, (TN, K, C), 1)
    pos_inf = jnp.float32(jnp.inf)
    min_sum = jnp.zeros((TN, C), dtype=jnp.float32)
    for _ in range(K - K2):
        m = jnp.min(knn, axis=1, keepdims=True)
        amin = jnp.min(jnp.where(knn == m, kk_iota, K),
                       axis=1, keepdims=True)
        min_sum = min_sum + m[:, 0, :]
        knn = jnp.where(kk_iota == amin, pos_inf, knn)
    x1_ref[0] = (s20 - min_sum) * jnp.float32(1.0 / K2)


def _feature_body(idx_ref, x1_full_ref, xt_tile_ref, out_ref):
    idx_tile = idx_ref[0]
    x1_full = x1_full_ref[0]
    xt_tile = xt_tile_ref[0]

    oh_all = (lax.broadcasted_iota(jnp.int32, (TN, K, N), 2)
              == idx_tile[:, :, None]).astype(jnp.float32).reshape(TN * K, N)
    gall = jnp.dot(oh_all, x1_full, precision=HIGHEST)
    g3 = gall.reshape(TN, K, C)
    parts = []
    for kk in range(K):
        parts.append(jnp.concatenate([g3[:, kk, :] - xt_tile, xt_tile],
                                     axis=1))
    out_ref[0] = jnp.concatenate(parts, axis=1)


@jax.jit
def _run(x):
    xt = jnp.transpose(x, (0, 2, 1))
    grid = (B, N // TN)
    idx, x1 = pl.pallas_call(
        _knn_body,
        grid=grid,
        in_specs=[
            pl.BlockSpec((1, TN, C), lambda b, i: (b, i, 0)),
            pl.BlockSpec((1, N, C), lambda b, i: (b, 0, 0)),
            pl.BlockSpec((1, C, N), lambda b, i: (b, 0, 0)),
        ],
        out_specs=[
            pl.BlockSpec((1, TN, K), lambda b, i: (b, i, 0)),
            pl.BlockSpec((1, TN, C), lambda b, i: (b, i, 0)),
        ],
        out_shape=[
            jax.ShapeDtypeStruct((B, N, K), jnp.int32),
            jax.ShapeDtypeStruct((B, N, C), jnp.float32),
        ],
    )(xt, xt, x)

    f2 = pl.pallas_call(
        _feature_body,
        grid=grid,
        in_specs=[
            pl.BlockSpec((1, TN, K), lambda b, i: (b, i, 0)),
            pl.BlockSpec((1, N, C), lambda b, i: (b, 0, 0)),
            pl.BlockSpec((1, TN, C), lambda b, i: (b, i, 0)),
        ],
        out_specs=pl.BlockSpec((1, TN, K * 2 * C), lambda b, i: (b, i, 0)),
        out_shape=jax.ShapeDtypeStruct((B, N, K * 2 * C), jnp.float32),
    )(idx, x1, xt)
    feature = jnp.transpose(f2.reshape(B, N, K, 2 * C), (0, 3, 1, 2))
    return feature, idx


def kernel(x, k, local_idx):
    feature, idx = _run(x)
    idx = idx + (jnp.asarray(k, idx.dtype) - K)
    idx_base = jnp.arange(B, dtype=idx.dtype).reshape(-1, 1, 1) * N
    idx_flat = (idx + idx_base).reshape(-1)
    return feature, idx_flat


# speedup vs baseline: 1.2615x; 1.2615x over previous
import functools

import jax
import jax.numpy as jnp
from jax import lax
from jax.experimental import pallas as pl
from jax.experimental.pallas import tpu as pltpu
from jax.experimental.pallas import tpu_sc as plsc

B, C, N = 8, 128, 1024
K = 20
K2 = 14
TN = 128
HIGHEST = lax.Precision.HIGHEST

NW = 32
NPW = (B * N) // NW
CN = 4
CHUNKS = NPW // CN


def _knn_body(xt_tile_ref, xt_full_ref, x_full_ref, idx_ref, x1_ref):
    xt_tile = xt_tile_ref[0]
    xt_full = xt_full_ref[0]
    x_full = x_full_ref[0]

    xx = jnp.sum(x_full * x_full, axis=0, keepdims=True)
    dist = 2.0 * jnp.dot(xt_tile, x_full) - xx

    lane_iota = lax.broadcasted_iota(jnp.int32, (TN, N), 1)
    neg_inf = jnp.float32(-jnp.inf)

    idx_cols = []
    for _ in range(K):
        m = jnp.max(dist, axis=1, keepdims=True)
        amax = jnp.min(jnp.where(dist == m, lane_iota, N),
                       axis=1, keepdims=True)
        idx_cols.append(amax)
        dist = jnp.where(lane_iota == amax, neg_inf, dist)
    idx_tile = jnp.concatenate(idx_cols, axis=1)
    idx_ref[0] = idx_tile + pl.program_id(0) * N

    knn_parts = []
    s20 = jnp.zeros((TN, C), dtype=jnp.float32)
    for kk in range(K):
        oh = (lane_iota == idx_tile[:, kk:kk + 1]).astype(jnp.float32)
        g = jnp.dot(oh, xt_full, precision=HIGHEST)
        s20 = s20 + g
        knn_parts.append(g.reshape(TN, 1, C))
    knn = jnp.concatenate(knn_parts, axis=1)

    kk_iota = lax.broadcasted_iota(jnp.int32, (TN, K, C), 1)
    pos_inf = jnp.float32(jnp.inf)
    min_sum = jnp.zeros((TN, C), dtype=jnp.float32)
    for _ in range(K - K2):
        m = jnp.min(knn, axis=1, keepdims=True)
        amin = jnp.min(jnp.where(knn == m, kk_iota, K),
                       axis=1, keepdims=True)
        min_sum = min_sum + m[:, 0, :]
        knn = jnp.where(kk_iota == amin, pos_inf, knn)
    x1_ref[0] = (s20 - min_sum) * jnp.float32(1.0 / K2)


def _sc_feature(x1_hbm, gidx_hbm, xt_hbm, out_hbm,
                idx_v, rows_v, xt_v, out_v, sem):
    wid = lax.axis_index("s") * 2 + lax.axis_index("c")
    base0 = wid * NPW

    def chunk_body(ci, carry):
        nbase = base0 + ci * CN
        pltpu.sync_copy(gidx_hbm.at[pl.ds(nbase * K, CN * K)], idx_v)
        pltpu.async_copy(x1_hbm.at[idx_v], rows_v, sem).wait()
        pltpu.sync_copy(xt_hbm.at[pl.ds(nbase, CN)], xt_v)
        for i in range(CN):
            for kk in range(K):
                for v in range(C // 16):
                    g = rows_v[i * K + kk, pl.ds(v * 16, 16)]
                    xr = xt_v[i, pl.ds(v * 16, 16)]
                    out_v[i, pl.ds(kk * 2 * C + v * 16, 16)] = g - xr
                    out_v[i, pl.ds(kk * 2 * C + C + v * 16, 16)] = xr
        pltpu.sync_copy(out_v, out_hbm.at[pl.ds(nbase, CN)])
        return carry

    lax.fori_loop(0, CHUNKS, chunk_body, 0)


@jax.jit
def _run(x):
    xt = jnp.transpose(x, (0, 2, 1))
    grid = (B, N // TN)
    gidx, x1 = pl.pallas_call(
        _knn_body,
        grid=grid,
        in_specs=[
            pl.BlockSpec((1, TN, C), lambda b, i: (b, i, 0)),
            pl.BlockSpec((1, N, C), lambda b, i: (b, 0, 0)),
            pl.BlockSpec((1, C, N), lambda b, i: (b, 0, 0)),
        ],
        out_specs=[
            pl.BlockSpec((1, TN, K), lambda b, i: (b, i, 0)),
            pl.BlockSpec((1, TN, C), lambda b, i: (b, i, 0)),
        ],
        out_shape=[
            jax.ShapeDtypeStruct((B, N, K), jnp.int32),
            jax.ShapeDtypeStruct((B, N, C), jnp.float32),
        ],
    )(xt, xt, x)

    sc_feature = functools.partial(
        pl.kernel,
        mesh=plsc.VectorSubcoreMesh(core_axis_name="c", subcore_axis_name="s"),
        out_type=jax.ShapeDtypeStruct((B * N, K * 2 * C), jnp.float32),
        scratch_types=[
            pltpu.VMEM((CN * K,), jnp.int32),
            pltpu.VMEM((CN * K, C), jnp.float32),
            pltpu.VMEM((CN, C), jnp.float32),
            pltpu.VMEM((CN, K * 2 * C), jnp.float32),
            pltpu.SemaphoreType.DMA,
        ],
    )(_sc_feature)
    f2 = sc_feature(x1.reshape(B * N, C), gidx.reshape(B * N * K),
                    xt.reshape(B * N, C))
    feature = jnp.transpose(f2.reshape(B, N, K, 2 * C), (0, 3, 1, 2))
    return feature, gidx


def kernel(x, k, local_idx):
    feature, gidx = _run(x)
    idx_flat = (gidx + (jnp.asarray(k, gidx.dtype) - K)).reshape(-1)
    return feature, idx_flat


# speedup vs baseline: 1.4939x; 1.1843x over previous
import functools

import jax
import jax.numpy as jnp
from jax import lax
from jax.experimental import pallas as pl
from jax.experimental.pallas import tpu as pltpu
from jax.experimental.pallas import tpu_sc as plsc

B, C, N = 8, 128, 1024
K = 20
K2 = 14
TN = 128
HIGHEST = lax.Precision.HIGHEST

NW = 32
NPW = (B * N) // NW
CN = 4
CHUNKS = NPW // CN


def _knn_body(xt_tile_ref, xt_full_ref, x_full_ref, idx_ref, x1_ref):
    xt_tile = xt_tile_ref[0]
    xt_full = xt_full_ref[0]
    x_full = x_full_ref[0]

    xx = jnp.sum(x_full * x_full, axis=0, keepdims=True)
    dist = 2.0 * jnp.dot(xt_tile, x_full) - xx

    lane_iota = lax.broadcasted_iota(jnp.int32, (TN, N), 1)
    neg_inf = jnp.float32(-jnp.inf)

    idx_cols = []
    for _ in range(K):
        m = jnp.max(dist, axis=1, keepdims=True)
        amax = jnp.min(jnp.where(dist == m, lane_iota, N),
                       axis=1, keepdims=True)
        idx_cols.append(amax)
        dist = jnp.where(lane_iota == amax, neg_inf, dist)
    idx_tile = jnp.concatenate(idx_cols, axis=1)
    idx_ref[0] = idx_tile + pl.program_id(0) * N

    hi = xt_full.astype(jnp.bfloat16)
    r1 = xt_full - hi.astype(jnp.float32)
    mid = r1.astype(jnp.bfloat16)
    lo = (r1 - mid.astype(jnp.float32)).astype(jnp.bfloat16)
    knn_parts = []
    s20 = jnp.zeros((TN, C), dtype=jnp.float32)
    for kk in range(K):
        oh = (lane_iota == idx_tile[:, kk:kk + 1]).astype(jnp.bfloat16)
        g = (jnp.dot(oh, hi, preferred_element_type=jnp.float32)
             + jnp.dot(oh, mid, preferred_element_type=jnp.float32)
             + jnp.dot(oh, lo, preferred_element_type=jnp.float32))
        s20 = s20 + g
        knn_parts.append(g.reshape(TN, 1, C))
    knn = jnp.concatenate(knn_parts, axis=1)

    kk_iota = lax.broadcasted_iota(jnp.int32, (TN, K, C), 1)
    pos_inf = jnp.float32(jnp.inf)
    min_sum = jnp.zeros((TN, C), dtype=jnp.float32)
    for _ in range(K - K2):
        m = jnp.min(knn, axis=1, keepdims=True)
        amin = jnp.min(jnp.where(knn == m, kk_iota, K),
                       axis=1, keepdims=True)
        min_sum = min_sum + m[:, 0, :]
        knn = jnp.where(kk_iota == amin, pos_inf, knn)
    x1_ref[0] = (s20 - min_sum) * jnp.float32(1.0 / K2)


def _sc_feature(x1_hbm, gidx_hbm, xt_hbm, out_hbm,
                idx_v, rows_v, xt_v, out_v, sem):
    wid = lax.axis_index("s") * 2 + lax.axis_index("c")
    base0 = wid * NPW

    def chunk_body(ci, carry):
        nbase = base0 + ci * CN
        pltpu.sync_copy(gidx_hbm.at[pl.ds(nbase * K, CN * K)], idx_v)
        pltpu.async_copy(x1_hbm.at[idx_v], rows_v, sem).wait()
        pltpu.sync_copy(xt_hbm.at[pl.ds(nbase, CN)], xt_v)
        for i in range(CN):
            for kk in range(K):
                for v in range(C // 16):
                    g = rows_v[i * K + kk, pl.ds(v * 16, 16)]
                    xr = xt_v[i, pl.ds(v * 16, 16)]
                    out_v[i, pl.ds(kk * 2 * C + v * 16, 16)] = g - xr
                    out_v[i, pl.ds(kk * 2 * C + C + v * 16, 16)] = xr
        pltpu.sync_copy(out_v, out_hbm.at[pl.ds(nbase, CN)])
        return carry

    lax.fori_loop(0, CHUNKS, chunk_body, 0)


@jax.jit
def _run(x):
    xt = jnp.transpose(x, (0, 2, 1))
    grid = (B, N // TN)
    gidx, x1 = pl.pallas_call(
        _knn_body,
        grid=grid,
        in_specs=[
            pl.BlockSpec((1, TN, C), lambda b, i: (b, i, 0)),
            pl.BlockSpec((1, N, C), lambda b, i: (b, 0, 0)),
            pl.BlockSpec((1, C, N), lambda b, i: (b, 0, 0)),
        ],
        out_specs=[
            pl.BlockSpec((1, TN, K), lambda b, i: (b, i, 0)),
            pl.BlockSpec((1, TN, C), lambda b, i: (b, i, 0)),
        ],
        out_shape=[
            jax.ShapeDtypeStruct((B, N, K), jnp.int32),
            jax.ShapeDtypeStruct((B, N, C), jnp.float32),
        ],
    )(xt, xt, x)

    sc_feature = functools.partial(
        pl.kernel,
        mesh=plsc.VectorSubcoreMesh(core_axis_name="c", subcore_axis_name="s"),
        out_type=jax.ShapeDtypeStruct((B * N, K * 2 * C), jnp.float32),
        scratch_types=[
            pltpu.VMEM((CN * K,), jnp.int32),
            pltpu.VMEM((CN * K, C), jnp.float32),
            pltpu.VMEM((CN, C), jnp.float32),
            pltpu.VMEM((CN, K * 2 * C), jnp.float32),
            pltpu.SemaphoreType.DMA,
        ],
    )(_sc_feature)
    f2 = sc_feature(x1.reshape(B * N, C), gidx.reshape(B * N * K),
                    xt.reshape(B * N, C))
    feature = jnp.transpose(f2.reshape(B, N, K, 2 * C), (0, 3, 1, 2))
    return feature, gidx


def kernel(x, k, local_idx):
    feature, gidx = _run(x)
    idx_flat = (gidx + (jnp.asarray(k, gidx.dtype) - K)).reshape(-1)
    return feature, idx_flat


# speedup vs baseline: 1.6088x; 1.0769x over previous
import functools

import jax
import jax.numpy as jnp
from jax import lax
from jax.experimental import pallas as pl
from jax.experimental.pallas import tpu as pltpu
from jax.experimental.pallas import tpu_sc as plsc

B, C, N = 8, 128, 1024
K = 20
K2 = 14
TN = 256
HIGHEST = lax.Precision.HIGHEST

NW = 32
NPW = (B * N) // NW
CN = 4
CHUNKS = NPW // CN


def _knn_body(xt_tile_ref, xt_full_ref, x_full_ref, idx_ref, x1_ref):
    xt_tile = xt_tile_ref[0]
    xt_full = xt_full_ref[0]
    x_full = x_full_ref[0]

    xx = jnp.sum(x_full * x_full, axis=0, keepdims=True)
    dist = 2.0 * jnp.dot(xt_tile, x_full) - xx

    lane_iota = lax.broadcasted_iota(jnp.int32, (TN, N), 1)
    neg_inf = jnp.float32(-jnp.inf)

    idx_cols = []
    for _ in range(K):
        m = jnp.max(dist, axis=1, keepdims=True)
        amax = jnp.min(jnp.where(dist == m, lane_iota, N),
                       axis=1, keepdims=True)
        idx_cols.append(amax)
        dist = jnp.where(lane_iota == amax, neg_inf, dist)
    idx_tile = jnp.concatenate(idx_cols, axis=1)
    idx_ref[0] = idx_tile + pl.program_id(0) * N

    hi = xt_full.astype(jnp.bfloat16)
    r1 = xt_full - hi.astype(jnp.float32)
    mid = r1.astype(jnp.bfloat16)
    lo = (r1 - mid.astype(jnp.float32)).astype(jnp.bfloat16)
    knn_parts = []
    s20 = jnp.zeros((TN, C), dtype=jnp.float32)
    for kk in range(K):
        oh = (lane_iota == idx_tile[:, kk:kk + 1]).astype(jnp.bfloat16)
        g = (jnp.dot(oh, hi, preferred_element_type=jnp.float32)
             + jnp.dot(oh, mid, preferred_element_type=jnp.float32)
             + jnp.dot(oh, lo, preferred_element_type=jnp.float32))
        s20 = s20 + g
        knn_parts.append(g.reshape(TN, 1, C))
    knn = jnp.concatenate(knn_parts, axis=1)

    kk_iota = lax.broadcasted_iota(jnp.int32, (TN, K, C), 1)
    pos_inf = jnp.float32(jnp.inf)
    min_sum = jnp.zeros((TN, C), dtype=jnp.float32)
    for _ in range(K - K2):
        m = jnp.min(knn, axis=1, keepdims=True)
        amin = jnp.min(jnp.where(knn == m, kk_iota, K),
                       axis=1, keepdims=True)
        min_sum = min_sum + m[:, 0, :]
        knn = jnp.where(kk_iota == amin, pos_inf, knn)
    x1_ref[0] = (s20 - min_sum) * jnp.float32(1.0 / K2)


def _sc_feature(x1_hbm, gidx_hbm, xt_hbm, out_hbm,
                idx_v, rows_v, xt_v, out_v, sem):
    wid = lax.axis_index("s") * 2 + lax.axis_index("c")
    base0 = wid * NPW

    def chunk_body(ci, carry):
        nbase = base0 + ci * CN
        pltpu.sync_copy(gidx_hbm.at[pl.ds(nbase * K, CN * K)], idx_v)
        pltpu.async_copy(x1_hbm.at[idx_v], rows_v, sem).wait()
        pltpu.sync_copy(xt_hbm.at[pl.ds(nbase, CN)], xt_v)
        for i in range(CN):
            for kk in range(K):
                for v in range(C // 16):
                    g = rows_v[i * K + kk, pl.ds(v * 16, 16)]
                    xr = xt_v[i, pl.ds(v * 16, 16)]
                    out_v[i, pl.ds(kk * 2 * C + v * 16, 16)] = g - xr
                    out_v[i, pl.ds(kk * 2 * C + C + v * 16, 16)] = xr
        pltpu.sync_copy(out_v, out_hbm.at[pl.ds(nbase, CN)])
        return carry

    lax.fori_loop(0, CHUNKS, chunk_body, 0)


@jax.jit
def _run(x):
    xt = jnp.transpose(x, (0, 2, 1))
    grid = (B, N // TN)
    gidx, x1 = pl.pallas_call(
        _knn_body,
        grid=grid,
        in_specs=[
            pl.BlockSpec((1, TN, C), lambda b, i: (b, i, 0)),
            pl.BlockSpec((1, N, C), lambda b, i: (b, 0, 0)),
            pl.BlockSpec((1, C, N), lambda b, i: (b, 0, 0)),
        ],
        out_specs=[
            pl.BlockSpec((1, TN, K), lambda b, i: (b, i, 0)),
            pl.BlockSpec((1, TN, C), lambda b, i: (b, i, 0)),
        ],
        out_shape=[
            jax.ShapeDtypeStruct((B, N, K), jnp.int32),
            jax.ShapeDtypeStruct((B, N, C), jnp.float32),
        ],
    )(xt, xt, x)

    sc_feature = functools.partial(
        pl.kernel,
        mesh=plsc.VectorSubcoreMesh(core_axis_name="c", subcore_axis_name="s"),
        out_type=jax.ShapeDtypeStruct((B * N, K * 2 * C), jnp.float32),
        scratch_types=[
            pltpu.VMEM((CN * K,), jnp.int32),
            pltpu.VMEM((CN * K, C), jnp.float32),
            pltpu.VMEM((CN, C), jnp.float32),
            pltpu.VMEM((CN, K * 2 * C), jnp.float32),
            pltpu.SemaphoreType.DMA,
        ],
    )(_sc_feature)
    f2 = sc_feature(x1.reshape(B * N, C), gidx.reshape(B * N * K),
                    xt.reshape(B * N, C))
    feature = jnp.transpose(f2.reshape(B, N, K, 2 * C), (0, 3, 1, 2))
    return feature, gidx


def kernel(x, k, local_idx):
    feature, gidx = _run(x)
    idx_flat = (gidx + (jnp.asarray(k, gidx.dtype) - K)).reshape(-1)
    return feature, idx_flat
